# baseline (speedup 1.0000x reference)
import functools

import jax
import jax.numpy as jnp
from jax import lax
from jax.experimental import pallas as pl
from jax.experimental.pallas import tpu as pltpu
from jax.experimental.pallas import tpu_sc as plsc

N = 10000
E = 160000
F = 256
H = 256
C = 64
BN = 1000


def _mm1_body(x_ref, w_ref, o_ref):
    o_ref[...] = jnp.dot(x_ref[...], w_ref[...],
                         preferred_element_type=jnp.float32)


def _mm1(x, w):
    nb = N // BN
    return pl.pallas_call(
        _mm1_body,
        grid=(nb, 2),
        in_specs=[
            pl.BlockSpec((BN, F), lambda i, j: (i, 0)),
            pl.BlockSpec((F, 128), lambda i, j: (0, j)),
        ],
        out_specs=pl.BlockSpec((BN, 128), lambda i, j: (j * nb + i, 0)),
        out_shape=jax.ShapeDtypeStruct((2 * N, 128), jnp.float32),
    )(x, w)


def _mm_mid_body(s_lo_ref, s_hi_ref, b_ref, w_ref, o_ref):
    h_lo = jnp.maximum(s_lo_ref[...] + b_ref[0], 0.0)
    h_hi = jnp.maximum(s_hi_ref[...] + b_ref[1], 0.0)
    o_ref[...] = (
        jnp.dot(h_lo, w_ref[0:128, :], preferred_element_type=jnp.float32)
        + jnp.dot(h_hi, w_ref[128:256, :], preferred_element_type=jnp.float32)
    )


def _mm_mid(s_split, b2d, w):
    nb = N // BN
    return pl.pallas_call(
        _mm_mid_body,
        grid=(nb, 2),
        in_specs=[
            pl.BlockSpec((BN, 128), lambda i, j: (i, 0)),
            pl.BlockSpec((BN, 128), lambda i, j: (nb + i, 0)),
            pl.BlockSpec((2, 128), lambda i, j: (0, 0)),
            pl.BlockSpec((256, 128), lambda i, j: (0, j)),
        ],
        out_specs=pl.BlockSpec((BN, 128), lambda i, j: (j * nb + i, 0)),
        out_shape=jax.ShapeDtypeStruct((2 * N, 128), jnp.float32),
    )(s_split, s_split, b2d, w)


def _mm4_body(s_lo_ref, s_hi_ref, b_ref, w_ref, o_ref):
    h_lo = jnp.maximum(s_lo_ref[...] + b_ref[0], 0.0)
    h_hi = jnp.maximum(s_hi_ref[...] + b_ref[1], 0.0)
    o_ref[...] = (
        jnp.dot(h_lo, w_ref[0:128, :], preferred_element_type=jnp.float32)
        + jnp.dot(h_hi, w_ref[128:256, :], preferred_element_type=jnp.float32)
    )


def _mm4(s_split, b2d, w4):
    nb = N // BN
    return pl.pallas_call(
        _mm4_body,
        grid=(nb,),
        in_specs=[
            pl.BlockSpec((BN, 128), lambda i: (i, 0)),
            pl.BlockSpec((BN, 128), lambda i: (nb + i, 0)),
            pl.BlockSpec((2, 128), lambda i: (0, 0)),
            pl.BlockSpec((256, C), lambda i: (0, 0)),
        ],
        out_specs=pl.BlockSpec((BN, C), lambda i: (i, 0)),
        out_shape=jax.ShapeDtypeStruct((N, C), jnp.float32),
    )(s_split, s_split, b2d, w4)


def _softmax_body(p_ref, b_ref, o_ref):
    z = p_ref[0] + p_ref[1] + b_ref[...]
    m = jnp.max(z, axis=1, keepdims=True)
    zs = z - m
    lse = jnp.log(jnp.sum(jnp.exp(zs), axis=1, keepdims=True))
    o_ref[...] = zs - lse


def _log_softmax(p, b4):
    nb = N // BN
    return pl.pallas_call(
        _softmax_body,
        grid=(nb,),
        in_specs=[
            pl.BlockSpec((2, BN, C), lambda i: (0, i, 0)),
            pl.BlockSpec((1, C), lambda i: (0, 0)),
        ],
        out_specs=pl.BlockSpec((BN, C), lambda i: (i, 0)),
        out_shape=jax.ShapeDtypeStruct((N, C), jnp.float32),
    )(p, b4.reshape(1, C))


def _spmm_split(xw_split, src, dst, wgt):
    lo = jnp.take(xw_split[:N], src, axis=0) * wgt[:, None]
    hi = jnp.take(xw_split[N:], src, axis=0) * wgt[:, None]
    out_lo = jax.ops.segment_sum(lo, dst, num_segments=N)
    out_hi = jax.ops.segment_sum(hi, dst, num_segments=N)
    return jnp.concatenate([out_lo, out_hi], axis=0)


def _spmm_last(xw, src, dst, wgt):
    msgs = jnp.take(xw, src, axis=0) * wgt[:, None]
    s = jax.ops.segment_sum(msgs, dst, num_segments=N)
    return jnp.stack([s, jnp.zeros_like(s)])


def kernel(x, edge_index, edge_weight, W1, b1, W2, b2, W3, b3, W4, b4):
    src = edge_index[0].astype(jnp.int32)
    dst = edge_index[1].astype(jnp.int32)
    wgt = edge_weight.astype(jnp.float32)

    xw1 = _mm1(x, W1)
    s1 = _spmm_split(xw1, src, dst, wgt)
    xw2 = _mm_mid(s1, b1.reshape(2, 128), W2)
    s2 = _spmm_split(xw2, src, dst, wgt)
    xw3 = _mm_mid(s2, b2.reshape(2, 128), W3)
    s3 = _spmm_split(xw3, src, dst, wgt)
    xw4 = _mm4(s3, b3.reshape(2, 128), W4)
    s4 = _spmm_last(xw4, src, dst, wgt)
    return _log_softmax(s4, b4)


# speedup vs baseline: 2.8221x; 2.8221x over previous
import functools

import jax
import jax.numpy as jnp
from jax import lax
from jax.experimental import pallas as pl
from jax.experimental.pallas import tpu as pltpu
from jax.experimental.pallas import tpu_sc as plsc

N = 10000
E = 160000
F = 256
H = 256
C = 64
BN = 1000


def _mm1_body(x_ref, w_ref, o_ref):
    o_ref[...] = jnp.dot(x_ref[...], w_ref[...],
                         preferred_element_type=jnp.float32)


def _mm1(x, w):
    nb = N // BN
    return pl.pallas_call(
        _mm1_body,
        grid=(nb, 2),
        in_specs=[
            pl.BlockSpec((BN, F), lambda i, j: (i, 0)),
            pl.BlockSpec((F, 128), lambda i, j: (0, j)),
        ],
        out_specs=pl.BlockSpec((BN, 128), lambda i, j: (j * nb + i, 0)),
        out_shape=jax.ShapeDtypeStruct((2 * N, 128), jnp.float32),
    )(x, w)


def _mm_mid_body(s_lo_ref, s_hi_ref, b_ref, w_ref, o_ref):
    h_lo = jnp.maximum(s_lo_ref[...] + b_ref[0], 0.0)
    h_hi = jnp.maximum(s_hi_ref[...] + b_ref[1], 0.0)
    o_ref[...] = (
        jnp.dot(h_lo, w_ref[0:128, :], preferred_element_type=jnp.float32)
        + jnp.dot(h_hi, w_ref[128:256, :], preferred_element_type=jnp.float32)
    )


def _mm_mid(s_split, b2d, w):
    nb = N // BN
    return pl.pallas_call(
        _mm_mid_body,
        grid=(nb, 2),
        in_specs=[
            pl.BlockSpec((BN, 128), lambda i, j: (i, 0)),
            pl.BlockSpec((BN, 128), lambda i, j: (nb + i, 0)),
            pl.BlockSpec((2, 128), lambda i, j: (0, 0)),
            pl.BlockSpec((256, 128), lambda i, j: (0, j)),
        ],
        out_specs=pl.BlockSpec((BN, 128), lambda i, j: (j * nb + i, 0)),
        out_shape=jax.ShapeDtypeStruct((2 * N, 128), jnp.float32),
    )(s_split, s_split, b2d, w)


def _mm4_body(s_lo_ref, s_hi_ref, b_ref, w_ref, o_ref):
    h_lo = jnp.maximum(s_lo_ref[...] + b_ref[0], 0.0)
    h_hi = jnp.maximum(s_hi_ref[...] + b_ref[1], 0.0)
    o_ref[...] = (
        jnp.dot(h_lo, w_ref[0:128, :], preferred_element_type=jnp.float32)
        + jnp.dot(h_hi, w_ref[128:256, :], preferred_element_type=jnp.float32)
    )


def _mm4(s_split, b2d, w4p):
    nb = N // BN
    return pl.pallas_call(
        _mm4_body,
        grid=(nb,),
        in_specs=[
            pl.BlockSpec((BN, 128), lambda i: (i, 0)),
            pl.BlockSpec((BN, 128), lambda i: (nb + i, 0)),
            pl.BlockSpec((2, 128), lambda i: (0, 0)),
            pl.BlockSpec((256, 128), lambda i: (0, 0)),
        ],
        out_specs=pl.BlockSpec((BN, 128), lambda i: (i, 0)),
        out_shape=jax.ShapeDtypeStruct((N, 128), jnp.float32),
    )(s_split, s_split, b2d, w4p)


def _softmax_body(p_ref, b_ref, o_ref):
    z = p_ref[0, :, 0:C] + p_ref[1, :, 0:C] + b_ref[...]
    m = jnp.max(z, axis=1, keepdims=True)
    zs = z - m
    lse = jnp.log(jnp.sum(jnp.exp(zs), axis=1, keepdims=True))
    o_ref[...] = zs - lse


def _log_softmax(p, b4):
    nb = N // BN
    return pl.pallas_call(
        _softmax_body,
        grid=(nb,),
        in_specs=[
            pl.BlockSpec((2, BN, 128), lambda i: (0, i, 0)),
            pl.BlockSpec((1, C), lambda i: (0, 0)),
        ],
        out_specs=pl.BlockSpec((BN, C), lambda i: (i, 0)),
        out_shape=jax.ShapeDtypeStruct((N, C), jnp.float32),
    )(p, b4.reshape(1, C))


_MESH = plsc.VectorSubcoreMesh(core_axis_name="c", subcore_axis_name="s")
NS = 16
FILLERS = 10
STRIPE = N // FILLERS
K = 128
E_PAD = 163840
ZR = 200


def _zero_fill(zbuf, acc_sh, s, width):
    zvec = jnp.zeros((16,), jnp.float32)

    def zrow(r, _):
        for j in range(width // 16):
            zbuf[r, pl.ds(16 * j, 16)] = zvec
        return 0

    @pl.when(s < FILLERS)
    def _():
        lax.fori_loop(0, ZR, zrow, 0)
        for t in range(STRIPE // ZR):
            pltpu.sync_copy(zbuf, acc_sh.at[pl.ds(s * STRIPE + t * ZR, ZR)])


def _scale_rows(rows_v, wgt_v, k, width):
    def scale16(g, _):
        w16 = wgt_v[pl.ds(g * 16, 16)]
        for lane in range(16):
            w = jnp.full((16,), w16[lane], jnp.float32)
            e = g * 16 + lane
            for j in range(width // 16):
                sl = pl.ds(16 * j, 16)
                rows_v[e, sl] = rows_v[e, sl] * w
        return 0

    lax.fori_loop(0, k // 16, scale16, 0)


def _spmm_split_body(xw_hbm, srcx_hbm, dst_hbm, wgt_hbm, out_hbm,
                     idx_v, dst_v, wgt_v, rows_v, zbuf, acc_sh, sem):
    c = lax.axis_index("c")
    s = lax.axis_index("s")
    _zero_fill(zbuf, acc_sh, s, 128)
    plsc.subcore_barrier()

    ep = E_PAD // NS
    t0 = s * ep

    def chunk(g, _):
        base = t0 + g * K
        pltpu.sync_copy(srcx_hbm.at[pl.ds(c * E_PAD + base, K)], idx_v)
        pltpu.sync_copy(dst_hbm.at[pl.ds(base, K)], dst_v)
        pltpu.sync_copy(wgt_hbm.at[pl.ds(base, K)], wgt_v)
        pltpu.async_copy(xw_hbm.at[idx_v], rows_v, sem).wait()
        _scale_rows(rows_v, wgt_v, K, 128)
        pltpu.sync_copy(rows_v, acc_sh.at[dst_v], add=True)
        return 0

    lax.fori_loop(0, ep // K, chunk, 0)
    plsc.subcore_barrier()

    @pl.when(s < FILLERS)
    def _():
        pltpu.sync_copy(acc_sh.at[pl.ds(s * STRIPE, STRIPE)],
                        out_hbm.at[pl.ds(c * N + s * STRIPE, STRIPE)])


_spmm_split_call = functools.partial(
    pl.kernel,
    out_type=jax.ShapeDtypeStruct((2 * N, 128), jnp.float32),
    mesh=_MESH,
    scratch_types=[
        pltpu.VMEM((K,), jnp.int32),
        pltpu.VMEM((K,), jnp.int32),
        pltpu.VMEM((K,), jnp.float32),
        pltpu.VMEM((K, 128), jnp.float32),
        pltpu.VMEM((ZR, 128), jnp.float32),
        pltpu.VMEM_SHARED((N, 128), jnp.float32),
        pltpu.SemaphoreType.DMA,
    ],
)(_spmm_split_body)


def _spmm_split(xw_split, srcx, dst, wgt):
    return _spmm_split_call(xw_split, srcx, dst, wgt)


def _spmm_last_body(xw_hbm, src_hbm, dst_hbm, wgt_hbm, out_hbm,
                    idx_v, dst_v, wgt_v, rows_v, zbuf, acc_sh, sem):
    c = lax.axis_index("c")
    s = lax.axis_index("s")
    _zero_fill(zbuf, acc_sh, s, 128)
    plsc.subcore_barrier()

    ep = E_PAD // (2 * NS)
    t0 = c * (E_PAD // 2) + s * ep

    def chunk(g, _):
        base = t0 + g * K
        pltpu.sync_copy(src_hbm.at[pl.ds(base, K)], idx_v)
        pltpu.sync_copy(dst_hbm.at[pl.ds(base, K)], dst_v)
        pltpu.sync_copy(wgt_hbm.at[pl.ds(base, K)], wgt_v)
        pltpu.async_copy(xw_hbm.at[idx_v], rows_v, sem).wait()
        _scale_rows(rows_v, wgt_v, K, 128)
        pltpu.sync_copy(rows_v, acc_sh.at[dst_v], add=True)
        return 0

    lax.fori_loop(0, ep // K, chunk, 0)
    plsc.subcore_barrier()

    @pl.when(s < FILLERS)
    def _():
        pltpu.sync_copy(acc_sh.at[pl.ds(s * STRIPE, STRIPE)],
                        out_hbm.at[c, pl.ds(s * STRIPE, STRIPE)])


_spmm_last_call = functools.partial(
    pl.kernel,
    out_type=jax.ShapeDtypeStruct((2, N, 128), jnp.float32),
    mesh=_MESH,
    scratch_types=[
        pltpu.VMEM((K,), jnp.int32),
        pltpu.VMEM((K,), jnp.int32),
        pltpu.VMEM((K,), jnp.float32),
        pltpu.VMEM((K, 128), jnp.float32),
        pltpu.VMEM((ZR, 128), jnp.float32),
        pltpu.VMEM_SHARED((N, 128), jnp.float32),
        pltpu.SemaphoreType.DMA,
    ],
)(_spmm_last_body)


def _spmm_last(xw, src, dst, wgt):
    return _spmm_last_call(xw, src, dst, wgt)


def kernel(x, edge_index, edge_weight, W1, b1, W2, b2, W3, b3, W4, b4):
    src = edge_index[0].astype(jnp.int32)
    dst = edge_index[1].astype(jnp.int32)
    wgt = edge_weight.astype(jnp.float32)
    pad = E_PAD - E
    zpad_i = jnp.zeros((pad,), jnp.int32)
    src_p = jnp.concatenate([src, zpad_i])
    dst_p = jnp.concatenate([dst, zpad_i])
    wgt_p = jnp.concatenate([wgt, jnp.zeros((pad,), jnp.float32)])
    srcx = jnp.concatenate([src_p, src_p + N])

    xw1 = _mm1(x, W1)
    s1 = _spmm_split(xw1, srcx, dst_p, wgt_p)
    xw2 = _mm_mid(s1, b1.reshape(2, 128), W2)
    s2 = _spmm_split(xw2, srcx, dst_p, wgt_p)
    xw3 = _mm_mid(s2, b2.reshape(2, 128), W3)
    s3 = _spmm_split(xw3, srcx, dst_p, wgt_p)
    w4p = jnp.concatenate([W4, jnp.zeros((H, 128 - C), jnp.float32)], axis=1)
    xw4 = _mm4(s3, b3.reshape(2, 128), w4p)
    s4 = _spmm_last(xw4, src_p, dst_p, wgt_p)
    return _log_softmax(s4, b4)
